# baseline (speedup 1.0000x reference)
import functools

import jax
import jax.numpy as jnp
from jax import lax
from jax.experimental import pallas as pl
from jax.experimental.pallas import tpu as pltpu
from jax.experimental.pallas import tpu_sc as plsc

N_ENV = 512
BUF = 1024
N_OBS = 64
N_ACT = 16
BATCH = 256

B = N_ENV * BATCH
NC = 2
NS = 16
L = 16
NW = NC * NS
EPW = N_ENV // NW
GPE = BATCH // L
FBW = 8192
SPW = EPW * BATCH

_mesh = plsc.VectorSubcoreMesh(core_axis_name="c", subcore_axis_name="s")


@functools.partial(
    pl.kernel,
    mesh=_mesh,
    compiler_params=pltpu.CompilerParams(use_tc_tiling_on_sc=False,
                                         needs_layout_passes=False),
    out_type=(
        jax.ShapeDtypeStruct((8, 1024, 8, 128), jnp.float32),
        jax.ShapeDtypeStruct((2, 1024, 8, 128), jnp.float32),
        jax.ShapeDtypeStruct((8, 1024, 8, 128), jnp.float32),
        jax.ShapeDtypeStruct((B,), jnp.float32),
        jax.ShapeDtypeStruct((B,), jnp.int32),
        jax.ShapeDtypeStruct((B,), jnp.int32),
        jax.ShapeDtypeStruct((B,), jnp.int32),
    ),
    scratch_types=(
        pltpu.VMEM((2 * 2048,), jnp.int32),
        pltpu.VMEM((2 * 8192,), jnp.float32),
        pltpu.VMEM((2 * 8192,), jnp.int32),
        pltpu.VMEM((2 * 8192,), jnp.int32),
        pltpu.VMEM((SPW,), jnp.float32),
        pltpu.VMEM((SPW,), jnp.int32),
        pltpu.VMEM((SPW,), jnp.int32),
        pltpu.VMEM((SPW,), jnp.int32),
        pltpu.VMEM((2 * FBW,), jnp.float32),
        pltpu.VMEM((2 * FBW,), jnp.float32),
        pltpu.VMEM((2, 2, 8, 128), jnp.float32),
        pltpu.VMEM((2, 2, 8, 128), jnp.float32),
        pltpu.SemaphoreType.DMA,
        pltpu.SemaphoreType.DMA,
        pltpu.SemaphoreType.DMA,
        pltpu.SemaphoreType.DMA,
        pltpu.SemaphoreType.DMA,
    ),
)
def _sample(obs_h, nxt_h, act_h, rew_h, dns_h, trc_h, idx_h,
            obs_o, act_o, nxt_o, rew_o, dns_o, trc_o, ones_o,
            idx_s, rew_s, dns_s, trc_s,
            rew_v, dns_v, trc_v, ones_v,
            sbuf0, sbuf1, vbuf0, vbuf1,
            ssem0, ssem1, osem0, osem1, scsem):
    wid = lax.axis_index("s") * NC + lax.axis_index("c")
    e0 = wid * EPW
    eb0 = e0 // 8
    sbuf = (sbuf0, sbuf1)
    vbuf = (vbuf0, vbuf1)
    ssem = (ssem0, ssem1)
    osem = (osem0, osem1)

    pltpu.sync_copy(idx_h.at[pl.ds(eb0 * 2048, 2 * 2048)], idx_s)
    scalar_cps = [
        pltpu.make_async_copy(rew_h.at[pl.ds(eb0 * 8192, 2 * 8192)],
                              rew_s, scsem),
        pltpu.make_async_copy(dns_h.at[pl.ds(eb0 * 8192, 2 * 8192)],
                              dns_s, scsem),
        pltpu.make_async_copy(trc_h.at[pl.ds(eb0 * 8192, 2 * 8192)],
                              trc_s, scsem),
    ]
    for cp in scalar_cps:
        cp.start()

    def fill_ones(j, c):
        ones_v[pl.ds(j * L, L)] = jnp.ones((L,), jnp.int32)
        return c
    lax.fori_loop(0, SPW // L, fill_ones, 0)

    def tvec_base(ebl, er, j):
        off = ebl * 2048 + (j // 8) * 1024 + er * 128 + (j % 8) * L
        t = idx_s[pl.ds(off, L)]
        tb = jax.lax.shift_right_logical(t, 7)
        tc = jax.lax.bitwise_and(t, 127)
        return jax.lax.shift_left(tb, 10) + tc

    def stage_cp(src, e, fb_off, nfb_total, p):
        return pltpu.make_async_copy(
            src.at[pl.ds((e * nfb_total + fb_off) * FBW, 2 * FBW)],
            sbuf[p], ssem[p])

    def out_cp(dst, e, fb_off, p):
        return pltpu.make_async_copy(
            vbuf[p], dst.at[pl.ds(fb_off, 2), pl.ds(2 * e, 2)], osem[p])

    def do_gather(p, ebl, er):
        @plsc.parallel_loop(0, GPE, 1, unroll=2)
        def grp(j):
            base = tvec_base(ebl, er, j)
            sb = j // 8
            sc0 = (j % 8) * L
            idx_fr = [base + fr * 128 for fr in range(8)]
            for fb in range(2):
                sub = sbuf[p].at[pl.ds(fb * FBW, FBW)]
                for fr in range(8):
                    vbuf[p][fb, sb, fr, pl.ds(sc0, L)] = (
                        plsc.load_gather(sub, [idx_fr[fr]]))

    def phase4(src, dst):
        stage_cp(src, e0, 0, 8, 0).start()

        def body(k, carry):
            e = e0 + k
            ebl = k // 8
            er = lax.rem(k, 8)
            for s in range(4):
                p = s % 2
                if s < 3:
                    stage_cp(src, e, 2 * (s + 1), 8, 1 - p).start()
                else:
                    @pl.when(k < EPW - 1)
                    def _():
                        stage_cp(src, e + 1, 0, 8, 1 - p).start()
                stage_cp(src, e, 2 * s, 8, p).wait()
                if s >= 2:
                    out_cp(dst, e, 2 * (s - 2), p).wait()
                else:
                    @pl.when(k > 0)
                    def _():
                        out_cp(dst, e - 1, 2 * (s + 2), p).wait()
                do_gather(p, ebl, er)
                out_cp(dst, e, 2 * s, p).start()
            return carry
        lax.fori_loop(0, EPW, body, 0)
        out_cp(dst, e0 + EPW - 1, 4, 0).wait()
        out_cp(dst, e0 + EPW - 1, 6, 1).wait()

    def phase1(src, dst):
        stage_cp(src, e0, 0, 2, 0).start()

        def body(k, carry):
            iA = 2 * k
            iB = iA + 1
            eA = e0 + iA
            eB = e0 + iB
            stage_cp(src, eB, 0, 2, 1).start()
            stage_cp(src, eA, 0, 2, 0).wait()

            @pl.when(k > 0)
            def _():
                out_cp(dst, eA - 2, 0, 0).wait()
            do_gather(0, iA // 8, lax.rem(iA, 8))
            out_cp(dst, eA, 0, 0).start()

            @pl.when(k < EPW // 2 - 1)
            def _():
                stage_cp(src, eA + 2, 0, 2, 0).start()
            stage_cp(src, eB, 0, 2, 1).wait()

            @pl.when(k > 0)
            def _():
                out_cp(dst, eB - 2, 0, 1).wait()
            do_gather(1, iB // 8, lax.rem(iB, 8))
            out_cp(dst, eB, 0, 1).start()
            return carry
        lax.fori_loop(0, EPW // 2, body, 0)
        out_cp(dst, e0 + EPW - 2, 0, 0).wait()
        out_cp(dst, e0 + EPW - 1, 0, 1).wait()

    phase4(obs_h, obs_o)
    phase4(nxt_h, nxt_o)
    phase1(act_h, act_o)

    for cp in scalar_cps:
        cp.wait()

    def scal_env(i, carry):
        ebl = i // 8
        er = lax.rem(i, 8)

        def grp(j, c):
            base = tvec_base(ebl, er, j) + ebl * 8192 + er * 128
            o = i * BATCH + j * L
            rew_v[pl.ds(o, L)] = plsc.load_gather(rew_s, [base])
            dns_v[pl.ds(o, L)] = plsc.load_gather(dns_s, [base])
            trc_v[pl.ds(o, L)] = plsc.load_gather(trc_s, [base])
            return c
        lax.fori_loop(0, GPE, grp, 0)
        return carry
    lax.fori_loop(0, EPW, scal_env, 0)

    final_cps = [
        pltpu.make_async_copy(rew_v, rew_o.at[pl.ds(e0 * BATCH, SPW)],
                              scsem),
        pltpu.make_async_copy(dns_v, dns_o.at[pl.ds(e0 * BATCH, SPW)],
                              scsem),
        pltpu.make_async_copy(trc_v, trc_o.at[pl.ds(e0 * BATCH, SPW)],
                              scsem),
        pltpu.make_async_copy(ones_v, ones_o.at[pl.ds(e0 * BATCH, SPW)],
                              scsem),
    ]
    for cp in final_cps:
        cp.start()
    for cp in final_cps:
        cp.wait()


def kernel(observations, next_observations, actions, rewards, dones,
           truncations, indices):
    obs5 = (observations.transpose(0, 2, 1)
            .reshape(N_ENV, 8, 8, 8, 128).transpose(0, 1, 3, 2, 4)
            .reshape(-1))
    nxt5 = (next_observations.transpose(0, 2, 1)
            .reshape(N_ENV, 8, 8, 8, 128).transpose(0, 1, 3, 2, 4)
            .reshape(-1))
    act5 = (actions.transpose(0, 2, 1)
            .reshape(N_ENV, 2, 8, 8, 128).transpose(0, 1, 3, 2, 4)
            .reshape(-1))
    rew4 = rewards.reshape(64, 8, 8, 128).transpose(0, 2, 1, 3).reshape(-1)
    dns4 = dones.reshape(64, 8, 8, 128).transpose(0, 2, 1, 3).reshape(-1)
    trc4 = truncations.reshape(64, 8, 8, 128).transpose(0, 2, 1, 3).reshape(-1)
    idx4 = indices.reshape(64, 8, 2, 128).transpose(0, 2, 1, 3).reshape(-1)

    obs_t, act_t, nxt_t, rews, dns, trcs, ones = _sample(
        obs5, nxt5, act5, rew4, dns4, trc4, idx4)

    obs = obs_t.transpose(1, 3, 0, 2).reshape(B, N_OBS)
    nxt = nxt_t.transpose(1, 3, 0, 2).reshape(B, N_OBS)
    acts = act_t.transpose(1, 3, 0, 2).reshape(B, N_ACT)
    return (obs, acts, nxt, rews, dns, trcs, ones)


# speedup vs baseline: 1.0949x; 1.0949x over previous
import functools

import jax
import jax.numpy as jnp
from jax import lax
from jax.experimental import pallas as pl
from jax.experimental.pallas import tpu as pltpu
from jax.experimental.pallas import tpu_sc as plsc

N_ENV = 512
BUF = 1024
N_OBS = 64
N_ACT = 16
BATCH = 256

B = N_ENV * BATCH
NC = 2
NS = 16
L = 16
NW = NC * NS
EPW = N_ENV // NW
GPE = BATCH // L
FBW = 8192

_mesh = plsc.VectorSubcoreMesh(core_axis_name="c", subcore_axis_name="s")


@functools.partial(
    pl.kernel,
    mesh=_mesh,
    compiler_params=pltpu.CompilerParams(use_tc_tiling_on_sc=False,
                                         needs_layout_passes=False),
    out_type=(
        jax.ShapeDtypeStruct((8, 1024, 8, 128), jnp.float32),
        jax.ShapeDtypeStruct((2, 1024, 8, 128), jnp.float32),
        jax.ShapeDtypeStruct((8, 1024, 8, 128), jnp.float32),
        jax.ShapeDtypeStruct((B,), jnp.float32),
        jax.ShapeDtypeStruct((B,), jnp.int32),
        jax.ShapeDtypeStruct((B,), jnp.int32),
        jax.ShapeDtypeStruct((B,), jnp.int32),
    ),
    scratch_types=(
        pltpu.VMEM((2 * 2048,), jnp.int32),
        pltpu.VMEM((8192,), jnp.float32),
        pltpu.VMEM((8192,), jnp.int32),
        pltpu.VMEM((8192,), jnp.int32),
        pltpu.VMEM((EPW * BATCH,), jnp.float32),
        pltpu.VMEM((EPW * BATCH,), jnp.int32),
        pltpu.VMEM((EPW * BATCH,), jnp.int32),
        pltpu.VMEM((EPW * BATCH,), jnp.int32),
        pltpu.VMEM((4 * FBW,), jnp.float32),
        pltpu.VMEM((4 * FBW,), jnp.float32),
        pltpu.VMEM((4, 2, 8, 128), jnp.float32),
        pltpu.VMEM((4, 2, 8, 128), jnp.float32),
        pltpu.SemaphoreType.DMA,
        pltpu.SemaphoreType.DMA,
        pltpu.SemaphoreType.DMA,
        pltpu.SemaphoreType.DMA,
        pltpu.SemaphoreType.DMA,
    ),
)
def _sample(obs_h, nxt_h, act_h, rew_h, dns_h, trc_h, idx_h,
            obs_o, act_o, nxt_o, rew_o, dns_o, trc_o, ones_o,
            idx_s, rew_s, dns_s, trc_s,
            rew_v, dns_v, trc_v, ones_v,
            sbuf0, sbuf1, vbuf0, vbuf1,
            ssem0, ssem1, osem0, osem1, scsem):
    wid = lax.axis_index("s") * NC + lax.axis_index("c")
    e0 = wid * EPW
    eb0 = e0 // 8
    sbuf = (sbuf0, sbuf1)
    vbuf = (vbuf0, vbuf1)
    ssem = (ssem0, ssem1)
    osem = (osem0, osem1)

    pltpu.sync_copy(idx_h.at[pl.ds(eb0 * 2048, 2 * 2048)], idx_s)

    def scal_stage_cps(ebl):
        return [
            pltpu.make_async_copy(
                src.at[pl.ds((eb0 + ebl) * 8192, 8192)], buf, scsem)
            for src, buf in ((rew_h, rew_s), (dns_h, dns_s), (trc_h, trc_s))
        ]

    for cp in scal_stage_cps(0):
        cp.start()

    def fill_ones(j, c):
        ones_v[pl.ds(j * L, L)] = jnp.ones((L,), jnp.int32)
        return c
    lax.fori_loop(0, EPW * BATCH // L, fill_ones, 0)

    def tvec_base(ebl, er, j):
        off = ebl * 2048 + (j // 8) * 1024 + er * 128 + (j % 8) * L
        t = idx_s[pl.ds(off, L)]
        tb = jax.lax.shift_right_logical(t, 7)
        tc = jax.lax.bitwise_and(t, 127)
        return jax.lax.shift_left(tb, 10) + tc

    def scal_gather(ebl):
        for cp in scal_stage_cps(ebl):
            cp.wait()

        def env_body(er, carry):
            def grp(j, c):
                base = tvec_base(ebl, er, j) + er * 128
                o = (ebl * 8 + er) * BATCH + j * L
                rew_v[pl.ds(o, L)] = plsc.load_gather(rew_s, [base])
                dns_v[pl.ds(o, L)] = plsc.load_gather(dns_s, [base])
                trc_v[pl.ds(o, L)] = plsc.load_gather(trc_s, [base])
                return c
            lax.fori_loop(0, GPE, grp, 0)
            return carry
        lax.fori_loop(0, 8, env_body, 0)

    def stage_cp(src, e, fb_off, nfb_total, nfb, p):
        return pltpu.make_async_copy(
            src.at[pl.ds((e * nfb_total + fb_off) * FBW, nfb * FBW)],
            sbuf[p].at[pl.ds(0, nfb * FBW)], ssem[p])

    def out_cp(dst, e, fb_off, nfb, p):
        return pltpu.make_async_copy(
            vbuf[p].at[pl.ds(0, nfb)],
            dst.at[pl.ds(fb_off, nfb), pl.ds(2 * e, 2)], osem[p])

    def do_gather(p, ebl, er, nfb):
        @plsc.parallel_loop(0, GPE, 1, unroll=2)
        def grp(j):
            base = tvec_base(ebl, er, j)
            sb = j // 8
            sc0 = (j % 8) * L
            idx_fr = [base + fr * 128 for fr in range(8)]
            for fb in range(nfb):
                sub = sbuf[p].at[pl.ds(fb * FBW, FBW)]
                for fr in range(8):
                    vbuf[p][fb, sb, fr, pl.ds(sc0, L)] = (
                        plsc.load_gather(sub, [idx_fr[fr]]))

    def phase2(src, dst):
        stage_cp(src, e0, 0, 8, 4, 0).start()

        def body(k, carry):
            e = e0 + k
            ebl = k // 8
            er = lax.rem(k, 8)
            stage_cp(src, e, 4, 8, 4, 1).start()
            stage_cp(src, e, 0, 8, 4, 0).wait()

            @pl.when(k > 0)
            def _():
                out_cp(dst, e - 1, 0, 4, 0).wait()
            do_gather(0, ebl, er, 4)
            out_cp(dst, e, 0, 4, 0).start()

            @pl.when(k < EPW - 1)
            def _():
                stage_cp(src, e + 1, 0, 8, 4, 0).start()
            stage_cp(src, e, 4, 8, 4, 1).wait()

            @pl.when(k > 0)
            def _():
                out_cp(dst, e - 1, 4, 4, 1).wait()
            do_gather(1, ebl, er, 4)
            out_cp(dst, e, 4, 4, 1).start()
            return carry
        lax.fori_loop(0, EPW, body, 0)
        out_cp(dst, e0 + EPW - 1, 0, 4, 0).wait()
        out_cp(dst, e0 + EPW - 1, 4, 4, 1).wait()

    def phase1(src, dst):
        stage_cp(src, e0, 0, 2, 2, 0).start()

        def body(k, carry):
            iA = 2 * k
            iB = iA + 1
            eA = e0 + iA
            eB = e0 + iB
            stage_cp(src, eB, 0, 2, 2, 1).start()
            stage_cp(src, eA, 0, 2, 2, 0).wait()

            @pl.when(k > 0)
            def _():
                out_cp(dst, eA - 2, 0, 2, 0).wait()
            do_gather(0, iA // 8, lax.rem(iA, 8), 2)
            out_cp(dst, eA, 0, 2, 0).start()

            @pl.when(k < EPW // 2 - 1)
            def _():
                stage_cp(src, eA + 2, 0, 2, 2, 0).start()
            stage_cp(src, eB, 0, 2, 2, 1).wait()

            @pl.when(k > 0)
            def _():
                out_cp(dst, eB - 2, 0, 2, 1).wait()
            do_gather(1, iB // 8, lax.rem(iB, 8), 2)
            out_cp(dst, eB, 0, 2, 1).start()
            return carry
        lax.fori_loop(0, EPW // 2, body, 0)
        out_cp(dst, e0 + EPW - 2, 0, 2, 0).wait()
        out_cp(dst, e0 + EPW - 1, 0, 2, 1).wait()

    phase2(obs_h, obs_o)
    scal_gather(0)
    for cp in scal_stage_cps(1):
        cp.start()
    phase2(nxt_h, nxt_o)
    scal_gather(1)
    final_cps = [
        pltpu.make_async_copy(buf, dst.at[pl.ds(e0 * BATCH, EPW * BATCH)],
                              scsem)
        for buf, dst in ((rew_v, rew_o), (dns_v, dns_o),
                         (trc_v, trc_o), (ones_v, ones_o))
    ]
    for cp in final_cps:
        cp.start()
    phase1(act_h, act_o)
    for cp in final_cps:
        cp.wait()


def kernel(observations, next_observations, actions, rewards, dones,
           truncations, indices):
    obs5 = (observations.transpose(0, 2, 1)
            .reshape(N_ENV, 8, 8, 8, 128).transpose(0, 1, 3, 2, 4)
            .reshape(-1))
    nxt5 = (next_observations.transpose(0, 2, 1)
            .reshape(N_ENV, 8, 8, 8, 128).transpose(0, 1, 3, 2, 4)
            .reshape(-1))
    act5 = (actions.transpose(0, 2, 1)
            .reshape(N_ENV, 2, 8, 8, 128).transpose(0, 1, 3, 2, 4)
            .reshape(-1))
    rew4 = rewards.reshape(64, 8, 8, 128).transpose(0, 2, 1, 3).reshape(-1)
    dns4 = dones.reshape(64, 8, 8, 128).transpose(0, 2, 1, 3).reshape(-1)
    trc4 = truncations.reshape(64, 8, 8, 128).transpose(0, 2, 1, 3).reshape(-1)
    idx4 = indices.reshape(64, 8, 2, 128).transpose(0, 2, 1, 3).reshape(-1)

    obs_t, act_t, nxt_t, rews, dns, trcs, ones = _sample(
        obs5, nxt5, act5, rew4, dns4, trc4, idx4)

    obs = obs_t.transpose(1, 3, 0, 2).reshape(B, N_OBS)
    nxt = nxt_t.transpose(1, 3, 0, 2).reshape(B, N_OBS)
    acts = act_t.transpose(1, 3, 0, 2).reshape(B, N_ACT)
    return (obs, acts, nxt, rews, dns, trcs, ones)
